# Initial kernel scaffold; baseline (speedup 1.0000x reference)
import jax
import jax.numpy as jnp
from jax.experimental import pallas as pl


def kernel(x, emb, W, b):
    raise NotImplementedError("write your pallas kernel here")


# speedup vs baseline: 181.4485x; 181.4485x over previous
import functools

import jax
import jax.numpy as jnp
from jax import lax
from jax.experimental import pallas as pl
from jax.experimental.pallas import tpu as pltpu
from jax.experimental.pallas import tpu_sc as plsc

B = 16384
L = 200
VOCAB = 1000
D = 64
VPAD = 1024

NC = 2
NS = 16
NW = NC * NS
LANES = 16

ROWS_PER_W = B // NW
GROUPS = ROWS_PER_W // LANES


def _table_kernel(emb_ref, w_ref, b_ref, s_ref):
    prod = emb_ref[...] * w_ref[...][None, None, :]
    s = jnp.sum(prod, axis=2)
    s_ref[...] = (s + b_ref[0]) * (1.0 / L)


def _pool_body(x_hbm, s_hbm, out_hbm, x_v, s_v, o_v):
    cid = lax.axis_index("c")
    sid = lax.axis_index("s")
    wid = sid * NC + cid
    base = wid * ROWS_PER_W

    pltpu.sync_copy(s_hbm, s_v)
    pltpu.sync_copy(x_hbm.at[pl.ds(base * L, ROWS_PER_W * L)], x_v)

    lane = lax.iota(jnp.int32, LANES)

    def group_body(g, carry):
        row0 = g * LANES
        xbase = (row0 + lane) * L

        def j_body(j, acc):
            xi = plsc.load_gather(x_v, [xbase + j])
            sv = plsc.load_gather(s_v, [xi])
            return acc + sv

        acc = lax.fori_loop(0, L, j_body, jnp.zeros((LANES,), jnp.float32),
                            unroll=8)
        res = 1.0 / (1.0 + jnp.exp(-acc))
        o_v[pl.ds(row0, LANES)] = res
        return carry

    lax.fori_loop(0, GROUPS, group_body, 0)
    pltpu.sync_copy(o_v, out_hbm.at[pl.ds(base, ROWS_PER_W)])


def kernel(x, emb, W, b):
    emb_p = jnp.zeros((VPAD, D), jnp.float32).at[:VOCAB].set(emb)
    emb3 = emb_p.reshape(8, 128, D)
    w = W.reshape(D).astype(jnp.float32)
    s2d = pl.pallas_call(
        _table_kernel,
        out_shape=jax.ShapeDtypeStruct((8, 128), jnp.float32),
    )(emb3, w, b.astype(jnp.float32))
    s_flat = s2d.reshape(VPAD)

    mesh = plsc.VectorSubcoreMesh(core_axis_name="c", subcore_axis_name="s")
    pool = functools.partial(
        pl.kernel,
        out_type=jax.ShapeDtypeStruct((B,), jnp.float32),
        mesh=mesh,
        scratch_types=[
            pltpu.VMEM((ROWS_PER_W * L,), jnp.int32),
            pltpu.VMEM((VPAD,), jnp.float32),
            pltpu.VMEM((ROWS_PER_W,), jnp.float32),
        ],
        compiler_params=pltpu.CompilerParams(needs_layout_passes=False),
    )(_pool_body)
    out = pool(x.reshape(B * L).astype(jnp.int32), s_flat)
    return out.reshape(B, 1)


# speedup vs baseline: 182.5749x; 1.0062x over previous
import functools

import jax
import jax.numpy as jnp
from jax import lax
from jax.experimental import pallas as pl
from jax.experimental.pallas import tpu as pltpu
from jax.experimental.pallas import tpu_sc as plsc

B = 16384
L = 200
VOCAB = 1000
D = 64

NC = 2
NS = 16
NW = NC * NS
LANES = 16

ROWS_PER_W = B // NW
GROUPS = ROWS_PER_W // LANES


def _table_kernel(emb_ref, w_ref, b_ref, s_ref):
    prod = emb_ref[...] * w_ref[...][None, :]
    s = jnp.sum(prod, axis=1)
    s_ref[...] = (s + b_ref[0]) * (1.0 / L)


def _pool_body(x_hbm, s_hbm, out_hbm, x_v, s_v, o_v):
    cid = lax.axis_index("c")
    sid = lax.axis_index("s")
    wid = sid * NC + cid
    base = wid * ROWS_PER_W

    pltpu.sync_copy(s_hbm, s_v)
    pltpu.sync_copy(x_hbm.at[pl.ds(base, ROWS_PER_W)], x_v)

    lane = lax.iota(jnp.int32, LANES)

    def group_body(g, carry):
        row0 = g * LANES
        rows = row0 + lane

        def j_body(j, acc):
            xi = plsc.load_gather(x_v, [rows, jnp.full((LANES,), j, jnp.int32)])
            sv = plsc.load_gather(s_v, [xi])
            return acc + sv

        acc = lax.fori_loop(0, L, j_body, jnp.zeros((LANES,), jnp.float32),
                            unroll=8)
        res = 1.0 / (1.0 + jnp.exp(-acc))
        o_v[pl.ds(row0, LANES)] = res
        return carry

    lax.fori_loop(0, GROUPS, group_body, 0)
    pltpu.sync_copy(o_v, out_hbm.at[pl.ds(base, ROWS_PER_W)])


def kernel(x, emb, W, b):
    w = W.reshape(D).astype(jnp.float32)
    s_flat = pl.pallas_call(
        _table_kernel,
        out_shape=jax.ShapeDtypeStruct((VOCAB,), jnp.float32),
    )(emb, w, b.astype(jnp.float32))

    mesh = plsc.VectorSubcoreMesh(core_axis_name="c", subcore_axis_name="s")
    pool = functools.partial(
        pl.kernel,
        out_type=jax.ShapeDtypeStruct((B,), jnp.float32),
        mesh=mesh,
        scratch_types=[
            pltpu.VMEM((ROWS_PER_W, L), jnp.int32),
            pltpu.VMEM((VOCAB,), jnp.float32),
            pltpu.VMEM((ROWS_PER_W,), jnp.float32),
        ],
        compiler_params=pltpu.CompilerParams(
            needs_layout_passes=False, use_tc_tiling_on_sc=False),
    )(_pool_body)
    out = pool(x.astype(jnp.int32), s_flat)
    return out.reshape(B, 1)
